# baseline (speedup 1.0000x reference)
import functools
import math

import jax
import jax.numpy as jnp
from jax import lax
from jax.experimental import pallas as pl
from jax.experimental.pallas import tpu as pltpu
from jax.experimental.pallas import tpu_sc as plsc

D = 64
SCALE = math.sqrt(D)
TPAD = 129

NC, NS = 2, 16
NW = NC * NS
BATCH, TLEN = 4096, 200
NBB = BATCH // 128
NTT = TLEN // 8
VBLK = 1000000 // 128

_mesh = plsc.VectorSubcoreMesh(core_axis_name="c", subcore_axis_name="s")


@functools.partial(
    pl.kernel,
    mesh=_mesh,
    compiler_params=pltpu.CompilerParams(
        use_tc_tiling_on_sc=True, needs_layout_passes=False
    ),
    out_type=jax.ShapeDtypeStruct((500000, 128), jnp.float32),
    scratch_types=[
        pltpu.VMEM((2, D, 128), jnp.float32),
        pltpu.VMEM((2, D, 128), jnp.float32),
        pltpu.VMEM((D, D), jnp.float32),
        pltpu.SemaphoreType.DMA,
        pltpu.SemaphoreType.DMA,
    ],
)
def _prep(tabT_hbm, tail_hbm, out_hbm, vin_v, vout_v, tail_v, isem, osem):
    w = lax.axis_index("s") * NC + lax.axis_index("c")
    lane = lax.iota(jnp.int32, 16)
    base_g = [lane + 16 * g for g in range(8)]
    row_g = [lane // 2 + 8 * g for g in range(8)]
    o64 = (lane % 2) * 64

    lo = w * VBLK // NW
    hi = (w + 1) * VBLK // NW

    def load(c, par):
        pltpu.async_copy(
            tabT_hbm.at[:, pl.ds(c * 128, 128)], vin_v.at[par], isem
        )

    def transpose(src, dst, ngroups):
        @plsc.parallel_loop(0, D, unroll=8)
        def _(d):
            for g in range(ngroups):
                vals = src[d, pl.ds(g * 16, 16)] * SCALE
                col = ((base_g[g] + d) & 63) + o64
                plsc.store_scatter(dst, [row_g[g], col], vals)

    def half(c, k, par):
        pltpu.make_async_copy(
            tabT_hbm.at[:, pl.ds(0, 128)], vin_v.at[par], isem
        ).wait()
        transpose(vin_v.at[par], vout_v.at[par], 8)

        @pl.when(c + 2 < hi)
        def _():
            load(c + 2, par)

        @pl.when(k >= 2)
        def _():
            pltpu.make_async_copy(
                vout_v.at[par], out_hbm.at[pl.ds((c - 2) * D, D)], osem
            ).wait()

        pltpu.async_copy(vout_v.at[par], out_hbm.at[pl.ds(c * D, D)], osem)

    load(lo, 0)
    load(lo + 1, 1)

    def pair(kp, carry):
        half(lo + 2 * kp, 2 * kp, 0)
        half(lo + 2 * kp + 1, 2 * kp + 1, 1)
        return carry

    n = hi - lo
    lax.fori_loop(0, n // 2, pair, 0)

    @pl.when(lax.rem(n, 2) == 1)
    def _():
        half(hi - 1, n - 1, 0)

    for back in (2, 1):
        pltpu.make_async_copy(
            vout_v.at[0], out_hbm.at[pl.ds((hi - back) * D, D)], osem
        ).wait()

    @pl.when(w == 0)
    def _():
        pltpu.sync_copy(tail_hbm, tail_v)
        transpose(tail_v, vout_v.at[0], 4)
        pltpu.sync_copy(
            vout_v.at[0, pl.ds(0, 32)], out_hbm.at[pl.ds(VBLK * D, 32)]
        )


@functools.partial(
    pl.kernel,
    mesh=_mesh,
    compiler_params=pltpu.CompilerParams(
        use_tc_tiling_on_sc=False, needs_layout_passes=False
    ),
    out_type=jax.ShapeDtypeStruct((TLEN, D // 8, NBB, 8, 128), jnp.float32),
    scratch_types=[
        pltpu.VMEM((NTT, 8, 128), jnp.int32),
        pltpu.VMEM((2, 128, D), jnp.float32),
        pltpu.VMEM((2, D, TPAD), jnp.float32),
        pltpu.SemaphoreType.DMA,
        pltpu.SemaphoreType.DMA,
    ],
)
def _embed(tok_hbm, table_hbm, out_hbm, tok_v, rows_v, tout_v, gsem, osem):
    j = lax.axis_index("s") * NC + lax.axis_index("c")
    lane = lax.iota(jnp.int32, 16)
    base_g = [lane + 16 * g for g in range(D // 16)]

    pltpu.sync_copy(tok_hbm.at[:, j], tok_v)
    pltpu.async_copy(table_hbm.at[tok_v.at[0, 0]], rows_v.at[0], gsem)
    pltpu.async_copy(table_hbm.at[tok_v.at[0, 1]], rows_v.at[1], gsem)

    def half_step(t, par):
        tt, s = t // 8, t % 8
        pltpu.make_async_copy(
            table_hbm.at[tok_v.at[0, 0]], rows_v.at[par], gsem
        ).wait()

        @plsc.parallel_loop(0, 128, unroll=16)
        def _(i):
            col = jnp.full((16,), i, jnp.int32)
            v = plsc.load_gather(tok_v.at[tt, s], [col])
            for g in range(D // 16):
                vals = rows_v[par, i, pl.ds(g * 16, 16)]
                dvec = (base_g[g] - v) & 63
                plsc.store_scatter(tout_v.at[par], [dvec, col], vals)

        @pl.when(t < TLEN - 2)
        def _():
            tn = t + 2
            pltpu.async_copy(
                table_hbm.at[tok_v.at[tn // 8, tn % 8]], rows_v.at[par], gsem
            )

        @pl.when(t >= 2)
        def _():
            for dt in range(D // 8):
                pltpu.make_async_copy(
                    tout_v.at[par, pl.ds(dt * 8, 8), pl.ds(0, 128)],
                    out_hbm.at[t - 2, dt, j],
                    osem,
                ).wait()

        for dt in range(D // 8):
            pltpu.async_copy(
                tout_v.at[par, pl.ds(dt * 8, 8), pl.ds(0, 128)],
                out_hbm.at[t, dt, j],
                osem,
            )

    def t_pair(tp, carry):
        half_step(tp * 2, 0)
        half_step(tp * 2 + 1, 1)
        return carry

    lax.fori_loop(0, TLEN // 2, t_pair, 0)
    for t, par in ((TLEN - 2, 0), (TLEN - 1, 1)):
        for dt in range(D // 8):
            pltpu.make_async_copy(
                tout_v.at[par, pl.ds(dt * 8, 8), pl.ds(0, 128)],
                out_hbm.at[t, dt, j],
                osem,
            ).wait()


def kernel(tokens, table):
    tok4 = (
        tokens.astype(jnp.int32)
        .transpose(1, 0)
        .reshape(NTT, 8, NBB, 128)
        .transpose(0, 2, 1, 3)
    )
    tab_t = table.transpose(1, 0)
    tab_rm = _prep(tab_t, tab_t[:, VBLK * 128 :])
    out5 = _embed(tok4, tab_rm.reshape(1000000, D))
    return out5.transpose(2, 4, 0, 1, 3).reshape(BATCH, TLEN, D)


# speedup vs baseline: 1.0252x; 1.0252x over previous
import functools
import math

import jax
import jax.numpy as jnp
from jax import lax
from jax.experimental import pallas as pl
from jax.experimental.pallas import tpu as pltpu
from jax.experimental.pallas import tpu_sc as plsc

D = 64
SCALE = math.sqrt(D)
TPAD = 129

NC, NS = 2, 16
NW = NC * NS
BATCH, TLEN = 4096, 200
NBB = BATCH // 128
NTT = TLEN // 8
VBLK = 1000000 // 128

_mesh = plsc.VectorSubcoreMesh(core_axis_name="c", subcore_axis_name="s")


@functools.partial(
    pl.kernel,
    mesh=_mesh,
    compiler_params=pltpu.CompilerParams(
        use_tc_tiling_on_sc=True, needs_layout_passes=False
    ),
    out_type=jax.ShapeDtypeStruct((500000, 128), jnp.float32),
    scratch_types=[
        pltpu.VMEM((2, D, 128), jnp.float32),
        pltpu.VMEM((2, D, 128), jnp.float32),
        pltpu.VMEM((D, D), jnp.float32),
        pltpu.SemaphoreType.DMA,
        pltpu.SemaphoreType.DMA,
    ],
)
def _prep(tabT_hbm, tail_hbm, out_hbm, vin_v, vout_v, tail_v, isem, osem):
    w = lax.axis_index("s") * NC + lax.axis_index("c")
    lane = lax.iota(jnp.int32, 16)
    base_g = [lane + 16 * g for g in range(8)]
    row_g = [lane // 2 + 8 * g for g in range(8)]
    o64 = (lane % 2) * 64

    lo = w * VBLK // NW
    hi = (w + 1) * VBLK // NW

    def load(c, par):
        pltpu.async_copy(
            tabT_hbm.at[:, pl.ds(c * 128, 128)], vin_v.at[par], isem
        )

    def transpose(src, dst, ngroups):
        @plsc.parallel_loop(0, D, unroll=8)
        def _(d):
            for g in range(ngroups):
                vals = src[d, pl.ds(g * 16, 16)] * SCALE
                col = ((base_g[g] + d) & 63) + o64
                plsc.store_scatter(dst, [row_g[g], col], vals)

    def half(c, k, par):
        pltpu.make_async_copy(
            tabT_hbm.at[:, pl.ds(0, 128)], vin_v.at[par], isem
        ).wait()
        transpose(vin_v.at[par], vout_v.at[par], 8)

        @pl.when(c + 2 < hi)
        def _():
            load(c + 2, par)

        @pl.when(k >= 2)
        def _():
            pltpu.make_async_copy(
                vout_v.at[par], out_hbm.at[pl.ds((c - 2) * D, D)], osem
            ).wait()

        pltpu.async_copy(vout_v.at[par], out_hbm.at[pl.ds(c * D, D)], osem)

    load(lo, 0)
    load(lo + 1, 1)

    def pair(kp, carry):
        half(lo + 2 * kp, 2 * kp, 0)
        half(lo + 2 * kp + 1, 2 * kp + 1, 1)
        return carry

    n = hi - lo
    lax.fori_loop(0, n // 2, pair, 0)

    @pl.when(lax.rem(n, 2) == 1)
    def _():
        half(hi - 1, n - 1, 0)

    for back in (2, 1):
        pltpu.make_async_copy(
            vout_v.at[0], out_hbm.at[pl.ds((hi - back) * D, D)], osem
        ).wait()

    @pl.when(w == 0)
    def _():
        pltpu.sync_copy(tail_hbm, tail_v)
        transpose(tail_v, vout_v.at[0], 4)
        pltpu.sync_copy(
            vout_v.at[0, pl.ds(0, 32)], out_hbm.at[pl.ds(VBLK * D, 32)]
        )


@functools.partial(
    pl.kernel,
    mesh=_mesh,
    compiler_params=pltpu.CompilerParams(
        use_tc_tiling_on_sc=False, needs_layout_passes=False
    ),
    out_type=jax.ShapeDtypeStruct((TLEN, D // 8, NBB, 8, 128), jnp.float32),
    scratch_types=[
        pltpu.VMEM((NTT, 8, 128), jnp.int32),
        pltpu.VMEM((2, 128, D), jnp.float32),
        pltpu.VMEM((2, D, TPAD), jnp.float32),
        pltpu.SemaphoreType.DMA,
        pltpu.SemaphoreType.DMA,
    ],
)
def _embed(tok_hbm, table_hbm, out_hbm, tok_v, rows_v, tout_v, gsem, osem):
    j = lax.axis_index("s") * NC + lax.axis_index("c")
    lane = lax.iota(jnp.int32, 16)
    base_g = [lane + 16 * g for g in range(D // 16)]

    pltpu.sync_copy(tok_hbm.at[:, j], tok_v)
    pltpu.async_copy(table_hbm.at[tok_v.at[0, 0]], rows_v.at[0], gsem)
    pltpu.async_copy(table_hbm.at[tok_v.at[0, 1]], rows_v.at[1], gsem)

    def half_step(t, par):
        tt, s = t // 8, t % 8
        pltpu.make_async_copy(
            table_hbm.at[tok_v.at[0, 0]], rows_v.at[par], gsem
        ).wait()

        @plsc.parallel_loop(0, 128, unroll=8)
        def _(i):
            col = jnp.full((16,), i, jnp.int32)
            v = plsc.load_gather(tok_v.at[tt, s], [col])
            for g in range(D // 16):
                vals = rows_v[par, i, pl.ds(g * 16, 16)]
                dvec = (base_g[g] - v) & 63
                plsc.store_scatter(tout_v.at[par], [dvec, col], vals)

        @pl.when(t < TLEN - 2)
        def _():
            tn = t + 2
            pltpu.async_copy(
                table_hbm.at[tok_v.at[tn // 8, tn % 8]], rows_v.at[par], gsem
            )

        @pl.when(t >= 2)
        def _():
            for dt in range(D // 8):
                pltpu.make_async_copy(
                    tout_v.at[par, pl.ds(dt * 8, 8), pl.ds(0, 128)],
                    out_hbm.at[t - 2, dt, j],
                    osem,
                ).wait()

        for dt in range(D // 8):
            pltpu.async_copy(
                tout_v.at[par, pl.ds(dt * 8, 8), pl.ds(0, 128)],
                out_hbm.at[t, dt, j],
                osem,
            )

    def t_pair(tp, carry):
        half_step(tp * 2, 0)
        half_step(tp * 2 + 1, 1)
        return carry

    lax.fori_loop(0, TLEN // 2, t_pair, 0)
    for t, par in ((TLEN - 2, 0), (TLEN - 1, 1)):
        for dt in range(D // 8):
            pltpu.make_async_copy(
                tout_v.at[par, pl.ds(dt * 8, 8), pl.ds(0, 128)],
                out_hbm.at[t, dt, j],
                osem,
            ).wait()


def kernel(tokens, table):
    tok4 = (
        tokens.astype(jnp.int32)
        .transpose(1, 0)
        .reshape(NTT, 8, NBB, 128)
        .transpose(0, 2, 1, 3)
    )
    tab_t = table.transpose(1, 0)
    tab_rm = _prep(tab_t, tab_t[:, VBLK * 128 :])
    out5 = _embed(tok4, tab_rm.reshape(1000000, D))
    return out5.transpose(2, 4, 0, 1, 3).reshape(BATCH, TLEN, D)


# speedup vs baseline: 1.1042x; 1.0770x over previous
import functools
import math

import jax
import jax.numpy as jnp
from jax import lax
from jax.experimental import pallas as pl
from jax.experimental.pallas import tpu as pltpu
from jax.experimental.pallas import tpu_sc as plsc

D = 64
SCALE = math.sqrt(D)
TPAD = 129

NC, NS = 2, 16
NW = NC * NS
BATCH, TLEN = 4096, 200
NBB = BATCH // 128
NTT = TLEN // 8
VBLK = 1000000 // 128

_mesh = plsc.VectorSubcoreMesh(core_axis_name="c", subcore_axis_name="s")


@functools.partial(
    pl.kernel,
    mesh=_mesh,
    compiler_params=pltpu.CompilerParams(
        use_tc_tiling_on_sc=True, needs_layout_passes=False
    ),
    out_type=jax.ShapeDtypeStruct((500000, 128), jnp.float32),
    scratch_types=[
        pltpu.VMEM((2, D, 256), jnp.float32),
        pltpu.VMEM((2, 128, 128), jnp.float32),
        pltpu.VMEM((D, D), jnp.float32),
        pltpu.SemaphoreType.DMA,
        pltpu.SemaphoreType.DMA,
    ],
)
def _prep(tabT_hbm, tail_hbm, out_hbm, vin_v, vout_v, tail_v, isem, osem):
    w = lax.axis_index("s") * NC + lax.axis_index("c")
    lane = lax.iota(jnp.int32, 16)
    base_g = [lane + 16 * g for g in range(16)]
    row_g = [lane // 2 + 8 * g for g in range(16)]
    o64 = (lane % 2) * 64

    nb2 = VBLK // 2
    lo = w * nb2 // NW
    hi = (w + 1) * nb2 // NW

    def load(c, par):
        pltpu.async_copy(
            tabT_hbm.at[:, pl.ds(c * 256, 256)], vin_v.at[par], isem
        )

    def transpose(src, dst, ngroups):
        @plsc.parallel_loop(0, D, unroll=8)
        def _(d):
            for g in range(ngroups):
                vals = src[d, pl.ds(g * 16, 16)] * SCALE
                col = ((base_g[g] + d) & 63) + o64
                plsc.store_scatter(dst, [row_g[g], col], vals)

    def half(c, k, par):
        pltpu.make_async_copy(
            tabT_hbm.at[:, pl.ds(0, 256)], vin_v.at[par], isem
        ).wait()
        transpose(vin_v.at[par], vout_v.at[par], 16)

        @pl.when(c + 2 < hi)
        def _():
            load(c + 2, par)

        @pl.when(k >= 2)
        def _():
            pltpu.make_async_copy(
                vout_v.at[par], out_hbm.at[pl.ds((c - 2) * 128, 128)], osem
            ).wait()

        pltpu.async_copy(
            vout_v.at[par], out_hbm.at[pl.ds(c * 128, 128)], osem
        )

    load(lo, 0)
    load(lo + 1, 1)

    def pair(kp, carry):
        half(lo + 2 * kp, 2 * kp, 0)
        half(lo + 2 * kp + 1, 2 * kp + 1, 1)
        return carry

    n = hi - lo
    lax.fori_loop(0, n // 2, pair, 0)

    @pl.when(lax.rem(n, 2) == 1)
    def _():
        half(hi - 1, n - 1, 0)

    for back in (2, 1):
        pltpu.make_async_copy(
            vout_v.at[0], out_hbm.at[pl.ds((hi - back) * 128, 128)], osem
        ).wait()

    @pl.when(w == 0)
    def _():
        pltpu.sync_copy(tail_hbm, tail_v)
        transpose(tail_v, vout_v.at[0], 4)
        pltpu.sync_copy(
            vout_v.at[0, pl.ds(0, 32)], out_hbm.at[pl.ds(VBLK * D, 32)]
        )


@functools.partial(
    pl.kernel,
    mesh=_mesh,
    compiler_params=pltpu.CompilerParams(
        use_tc_tiling_on_sc=False, needs_layout_passes=False
    ),
    out_type=jax.ShapeDtypeStruct((TLEN, D // 8, NBB, 8, 128), jnp.float32),
    scratch_types=[
        pltpu.VMEM((NTT, 8, 128), jnp.int32),
        pltpu.VMEM((2, 128, D), jnp.float32),
        pltpu.VMEM((2, D, TPAD), jnp.float32),
        pltpu.SemaphoreType.DMA,
        pltpu.SemaphoreType.DMA,
    ],
)
def _embed(tok_hbm, table_hbm, out_hbm, tok_v, rows_v, tout_v, gsem, osem):
    j = lax.axis_index("s") * NC + lax.axis_index("c")
    lane = lax.iota(jnp.int32, 16)
    base_g = [lane + 16 * g for g in range(D // 16)]

    pltpu.sync_copy(tok_hbm.at[:, j], tok_v)
    pltpu.async_copy(table_hbm.at[tok_v.at[0, 0]], rows_v.at[0], gsem)
    pltpu.async_copy(table_hbm.at[tok_v.at[0, 1]], rows_v.at[1], gsem)

    def half_step(t, par):
        tt, s = t // 8, t % 8
        pltpu.make_async_copy(
            table_hbm.at[tok_v.at[0, 0]], rows_v.at[par], gsem
        ).wait()

        @plsc.parallel_loop(0, 128, unroll=8)
        def _(i):
            col = jnp.full((16,), i, jnp.int32)
            v = plsc.load_gather(tok_v.at[tt, s], [col])
            for g in range(D // 16):
                vals = rows_v[par, i, pl.ds(g * 16, 16)]
                dvec = (base_g[g] - v) & 63
                plsc.store_scatter(tout_v.at[par], [dvec, col], vals)

        @pl.when(t < TLEN - 2)
        def _():
            tn = t + 2
            pltpu.async_copy(
                table_hbm.at[tok_v.at[tn // 8, tn % 8]], rows_v.at[par], gsem
            )

        @pl.when(t >= 2)
        def _():
            for dt in range(D // 8):
                pltpu.make_async_copy(
                    tout_v.at[par, pl.ds(dt * 8, 8), pl.ds(0, 128)],
                    out_hbm.at[t - 2, dt, j],
                    osem,
                ).wait()

        for dt in range(D // 8):
            pltpu.async_copy(
                tout_v.at[par, pl.ds(dt * 8, 8), pl.ds(0, 128)],
                out_hbm.at[t, dt, j],
                osem,
            )

    def t_pair(tp, carry):
        half_step(tp * 2, 0)
        half_step(tp * 2 + 1, 1)
        return carry

    lax.fori_loop(0, TLEN // 2, t_pair, 0)
    for t, par in ((TLEN - 2, 0), (TLEN - 1, 1)):
        for dt in range(D // 8):
            pltpu.make_async_copy(
                tout_v.at[par, pl.ds(dt * 8, 8), pl.ds(0, 128)],
                out_hbm.at[t, dt, j],
                osem,
            ).wait()


def kernel(tokens, table):
    tok4 = (
        tokens.astype(jnp.int32)
        .transpose(1, 0)
        .reshape(NTT, 8, NBB, 128)
        .transpose(0, 2, 1, 3)
    )
    tab_t = table.transpose(1, 0)
    tab_rm = _prep(tab_t, tab_t[:, VBLK * 128 :])
    out5 = _embed(tok4, tab_rm.reshape(1000000, D))
    return out5.transpose(2, 4, 0, 1, 3).reshape(BATCH, TLEN, D)
